# baseline (speedup 1.0000x reference)
import functools

import jax
import jax.numpy as jnp
from jax.experimental import pallas as pl
from jax.experimental.pallas import tpu as pltpu

B, HD = 4, 128
N_ALL = B * HD
NBUF = 4


def _stage_body(adj_hbm, src_ref, hprev_ref,
                w1t_ref, b1_ref, w2t_ref, b2_ref,
                wiht_ref, bih_ref, whht_ref, bhh_ref,
                out_ref, outT_ref, buf, sems, *, tile_m):
    i = pl.program_id(0)
    n = pl.num_programs(0)

    def copy(chunk, slot):
        return pltpu.make_async_copy(
            adj_hbm.at[pl.ds(chunk * tile_m, tile_m), :],
            buf.at[slot], sems.at[slot])

    @pl.when(i == 0)
    def _():
        for d in range(min(NBUF, n)):
            copy(d, d).start()

    slot = jax.lax.rem(i, NBUF)
    copy(i, slot).wait()

    agg_all = jnp.dot(buf[slot].astype(jnp.bfloat16), src_ref[...],
                      preferred_element_type=jnp.float32)

    @pl.when(i + NBUF < n)
    def _():
        copy(i + NBUF, slot).start()

    w1t = w1t_ref[...]
    w2t = w2t_ref[...]
    wiht = wiht_ref[...]
    whht = whht_ref[...]
    b1 = b1_ref[...]
    b2 = b2_ref[...]
    bih = bih_ref[...]
    bhh = bhh_ref[...]
    for b in range(B):
        agg = agg_all[:, b * HD:(b + 1) * HD]
        h1 = jnp.dot(agg, w1t, preferred_element_type=jnp.float32) + b1
        g = 0.5 * h1 * (1.0 + jax.lax.erf(h1 * 0.7071067811865476))
        x = jnp.dot(g, w2t, preferred_element_type=jnp.float32) + b2
        hprev = hprev_ref[b]
        gi = jnp.dot(x, wiht, preferred_element_type=jnp.float32) + bih
        gh = jnp.dot(hprev, whht, preferred_element_type=jnp.float32) + bhh
        r = jax.nn.sigmoid(gi[:, :HD] + gh[:, :HD])
        z = jax.nn.sigmoid(gi[:, HD:2 * HD] + gh[:, HD:2 * HD])
        n_g = jnp.tanh(gi[:, 2 * HD:] + r * gh[:, 2 * HD:])
        new = (1.0 - z) * n_g + z * hprev
        out_ref[b] = new
        if outT_ref is not None:
            outT_ref[:, b * HD:(b + 1) * HD] = new.astype(jnp.bfloat16)


def _stage(adj, src, hprev, w1t, b1, w2t, b2, wiht, bih, whht, bhh,
           tile_m, emit_transposed):
    m, k = adj.shape
    grid = (m // tile_m,)
    in_specs = [
        pl.BlockSpec(memory_space=pl.ANY),
        pl.BlockSpec((k, N_ALL), lambda i: (0, 0)),
        pl.BlockSpec((B, tile_m, HD), lambda i: (0, i, 0)),
        pl.BlockSpec((HD, HD), lambda i: (0, 0)),
        pl.BlockSpec((1, HD), lambda i: (0, 0)),
        pl.BlockSpec((HD, HD), lambda i: (0, 0)),
        pl.BlockSpec((1, HD), lambda i: (0, 0)),
        pl.BlockSpec((HD, 3 * HD), lambda i: (0, 0)),
        pl.BlockSpec((1, 3 * HD), lambda i: (0, 0)),
        pl.BlockSpec((HD, 3 * HD), lambda i: (0, 0)),
        pl.BlockSpec((1, 3 * HD), lambda i: (0, 0)),
    ]
    out_shape = [jax.ShapeDtypeStruct((B, m, HD), jnp.float32)]
    out_specs = [pl.BlockSpec((B, tile_m, HD), lambda i: (0, i, 0))]
    if emit_transposed:
        out_shape.append(jax.ShapeDtypeStruct((m, N_ALL), jnp.bfloat16))
        out_specs.append(pl.BlockSpec((tile_m, N_ALL), lambda i: (i, 0)))
        body = functools.partial(_stage_body, tile_m=tile_m)
    else:
        def body(*refs):
            _stage_body(*refs[:-2], None, *refs[-2:], tile_m=tile_m)
    return pl.pallas_call(
        body,
        grid=grid,
        in_specs=in_specs,
        out_specs=out_specs,
        out_shape=out_shape,
        scratch_shapes=[
            pltpu.VMEM((NBUF, tile_m, k), jnp.float32),
            pltpu.SemaphoreType.DMA((NBUF,)),
        ],
        compiler_params=pltpu.CompilerParams(
            dimension_semantics=("arbitrary",),
            vmem_limit_bytes=64 * 1024 * 1024,
        ),
    )(adj, src, hprev, w1t, b1, w2t, b2, wiht, bih, whht, bhh)


def kernel(v_feats, c_feats, H, H_t, W1, b1, W2, b2,
           var_wih, var_whh, var_bih, var_bhh,
           chk_wih, chk_whh, chk_bih, chk_bhh):
    w1t = W1.T
    w2t = W2.T
    b1r = b1.reshape(1, HD)
    b2r = b2.reshape(1, HD)
    chk_wiht = chk_wih.T
    chk_whht = chk_whh.T
    var_wiht = var_wih.T
    var_whht = var_whh.T
    chk_bihr = chk_bih.reshape(1, 3 * HD)
    chk_bhhr = chk_bhh.reshape(1, 3 * HD)
    var_bihr = var_bih.reshape(1, 3 * HD)
    var_bhhr = var_bhh.reshape(1, 3 * HD)

    v_src = jnp.transpose(v_feats, (1, 0, 2)).reshape(-1, N_ALL).astype(jnp.bfloat16)
    c_new, c_newT = _stage(H, v_src, c_feats,
                           w1t, b1r, w2t, b2r,
                           chk_wiht, chk_bihr, chk_whht, chk_bhhr,
                           tile_m=200, emit_transposed=True)
    (v_new,) = _stage(H_t, c_newT, v_feats,
                      w1t, b1r, w2t, b2r,
                      var_wiht, var_bihr, var_whht, var_bhhr,
                      tile_m=400, emit_transposed=False)
    return (v_new, c_new)


# speedup vs baseline: 1.1122x; 1.1122x over previous
import jax
import jax.numpy as jnp
from jax.experimental import pallas as pl
from jax.experimental.pallas import tpu as pltpu

B, HD = 4, 128
N_ALL = B * HD


def _stage_body(adj_ref, src_ref, hprev_ref,
                w1t_ref, b1_ref, w2t_ref, b2_ref,
                wiht_ref, bih_ref, whht_ref, bhh_ref,
                out_ref, outT_ref, *, packed_h):
    agg_all = jnp.dot(adj_ref[...].astype(jnp.bfloat16), src_ref[...],
                      preferred_element_type=jnp.float32)
    w1t = w1t_ref[...]
    w2t = w2t_ref[...]
    wiht = wiht_ref[...]
    whht = whht_ref[...]
    b1 = b1_ref[...]
    b2 = b2_ref[...]
    bih = bih_ref[...]
    bhh = bhh_ref[...]
    for b in range(B):
        agg = agg_all[:, b * HD:(b + 1) * HD]
        h1 = jnp.dot(agg, w1t, preferred_element_type=jnp.float32) + b1
        g = 0.5 * h1 * (1.0 + jax.lax.erf(h1 * 0.7071067811865476))
        x = jnp.dot(g, w2t, preferred_element_type=jnp.float32) + b2
        if packed_h:
            hprev = hprev_ref[:, b * HD:(b + 1) * HD].astype(jnp.float32)
        else:
            hprev = hprev_ref[b]
        gi = jnp.dot(x, wiht, preferred_element_type=jnp.float32) + bih
        gh = jnp.dot(hprev, whht, preferred_element_type=jnp.float32) + bhh
        r = jax.nn.sigmoid(gi[:, :HD] + gh[:, :HD])
        z = jax.nn.sigmoid(gi[:, HD:2 * HD] + gh[:, HD:2 * HD])
        n = jnp.tanh(gi[:, 2 * HD:] + r * gh[:, 2 * HD:])
        new = (1.0 - z) * n + z * hprev
        out_ref[b] = new
        if outT_ref is not None:
            outT_ref[:, b * HD:(b + 1) * HD] = new.astype(jnp.bfloat16)


def _stage(adj, src, hprev, w1t, b1, w2t, b2, wiht, bih, whht, bhh,
           tile_m, emit_transposed, packed_h):
    m, k = adj.shape
    grid = (pl.cdiv(m, tile_m),)
    if packed_h:
        hprev_spec = pl.BlockSpec((tile_m, N_ALL), lambda i: (i, 0))
    else:
        hprev_spec = pl.BlockSpec((B, tile_m, HD), lambda i: (0, i, 0))
    in_specs = [
        pl.BlockSpec((tile_m, k), lambda i: (i, 0)),
        pl.BlockSpec((k, N_ALL), lambda i: (0, 0)),
        hprev_spec,
        pl.BlockSpec((HD, HD), lambda i: (0, 0)),
        pl.BlockSpec((1, HD), lambda i: (0, 0)),
        pl.BlockSpec((HD, HD), lambda i: (0, 0)),
        pl.BlockSpec((1, HD), lambda i: (0, 0)),
        pl.BlockSpec((HD, 3 * HD), lambda i: (0, 0)),
        pl.BlockSpec((1, 3 * HD), lambda i: (0, 0)),
        pl.BlockSpec((HD, 3 * HD), lambda i: (0, 0)),
        pl.BlockSpec((1, 3 * HD), lambda i: (0, 0)),
    ]
    out_shape = [jax.ShapeDtypeStruct((B, m, HD), jnp.float32)]
    out_specs = [pl.BlockSpec((B, tile_m, HD), lambda i: (0, i, 0))]
    if emit_transposed:
        out_shape.append(jax.ShapeDtypeStruct((m, N_ALL), jnp.bfloat16))
        out_specs.append(pl.BlockSpec((tile_m, N_ALL), lambda i: (i, 0)))

    def body(*refs):
        if emit_transposed:
            _stage_body(*refs, packed_h=packed_h)
        else:
            _stage_body(*refs, None, packed_h=packed_h)

    return pl.pallas_call(
        body,
        grid=grid,
        in_specs=in_specs,
        out_specs=out_specs,
        out_shape=out_shape,
        compiler_params=pltpu.CompilerParams(
            dimension_semantics=("parallel",),
            vmem_limit_bytes=64 * 1024 * 1024,
        ),
    )(adj, src, hprev, w1t, b1, w2t, b2, wiht, bih, whht, bhh)


def kernel(v_feats, c_feats, H, H_t, W1, b1, W2, b2,
           var_wih, var_whh, var_bih, var_bhh,
           chk_wih, chk_whh, chk_bih, chk_bhh):
    w1t = W1.T
    w2t = W2.T
    b1r = b1.reshape(1, HD)
    b2r = b2.reshape(1, HD)
    chk_wiht = chk_wih.T
    chk_whht = chk_whh.T
    var_wiht = var_wih.T
    var_whht = var_whh.T
    chk_bihr = chk_bih.reshape(1, 3 * HD)
    chk_bhhr = chk_bhh.reshape(1, 3 * HD)
    var_bihr = var_bih.reshape(1, 3 * HD)
    var_bhhr = var_bhh.reshape(1, 3 * HD)

    v_src = jnp.transpose(v_feats, (1, 0, 2)).reshape(-1, N_ALL).astype(jnp.bfloat16)
    c_new, c_newT = _stage(H, v_src, c_feats,
                           w1t, b1r, w2t, b2r,
                           chk_wiht, chk_bihr, chk_whht, chk_bhhr,
                           tile_m=512, emit_transposed=True, packed_h=False)
    (v_new,) = _stage(H_t, c_newT, v_src,
                      w1t, b1r, w2t, b2r,
                      var_wiht, var_bihr, var_whht, var_bhhr,
                      tile_m=1024, emit_transposed=False, packed_h=True)
    return (v_new, c_new)


# speedup vs baseline: 1.1169x; 1.0043x over previous
import jax
import jax.numpy as jnp
from jax.experimental import pallas as pl
from jax.experimental.pallas import tpu as pltpu

B, HD = 4, 128
N_ALL = B * HD


def _stage_body(adj_ref, src_ref, hprev_ref,
                w1t_ref, b1_ref, w2t_ref, b2_ref,
                wiht_ref, bih_ref, whht_ref, bhh_ref,
                out_ref, outT_ref, *, packed_h):
    agg_all = jnp.dot(adj_ref[...].astype(jnp.bfloat16), src_ref[...],
                      preferred_element_type=jnp.float32)
    w1t = w1t_ref[...]
    w2t = w2t_ref[...]
    wiht = wiht_ref[...]
    whht = whht_ref[...]
    b1 = b1_ref[...]
    b2 = b2_ref[...]
    bih = bih_ref[...]
    bhh = bhh_ref[...]
    for b in range(B):
        agg = agg_all[:, b * HD:(b + 1) * HD]
        h1 = jnp.dot(agg, w1t, preferred_element_type=jnp.float32) + b1
        g = 0.5 * h1 * (1.0 + jax.lax.erf(h1 * 0.7071067811865476))
        x = jnp.dot(g, w2t, preferred_element_type=jnp.float32) + b2
        if packed_h:
            hprev = hprev_ref[:, b * HD:(b + 1) * HD].astype(jnp.float32)
        else:
            hprev = hprev_ref[b]
        gi = jnp.dot(x, wiht, preferred_element_type=jnp.float32) + bih
        gh = jnp.dot(hprev, whht, preferred_element_type=jnp.float32) + bhh
        r = jax.nn.sigmoid(gi[:, :HD] + gh[:, :HD])
        z = jax.nn.sigmoid(gi[:, HD:2 * HD] + gh[:, HD:2 * HD])
        n = jnp.tanh(gi[:, 2 * HD:] + r * gh[:, 2 * HD:])
        new = (1.0 - z) * n + z * hprev
        out_ref[b] = new
        if outT_ref is not None:
            outT_ref[:, b * HD:(b + 1) * HD] = new.astype(jnp.bfloat16)


def _stage(adj, src, hprev, w1t, b1, w2t, b2, wiht, bih, whht, bhh,
           tile_m, emit_transposed, packed_h):
    m, k = adj.shape
    grid = (pl.cdiv(m, tile_m),)
    if packed_h:
        hprev_spec = pl.BlockSpec((tile_m, N_ALL), lambda i: (i, 0))
    else:
        hprev_spec = pl.BlockSpec((B, tile_m, HD), lambda i: (0, i, 0))
    in_specs = [
        pl.BlockSpec((tile_m, k), lambda i: (i, 0)),
        pl.BlockSpec((k, N_ALL), lambda i: (0, 0)),
        hprev_spec,
        pl.BlockSpec((HD, HD), lambda i: (0, 0)),
        pl.BlockSpec((1, HD), lambda i: (0, 0)),
        pl.BlockSpec((HD, HD), lambda i: (0, 0)),
        pl.BlockSpec((1, HD), lambda i: (0, 0)),
        pl.BlockSpec((HD, 3 * HD), lambda i: (0, 0)),
        pl.BlockSpec((1, 3 * HD), lambda i: (0, 0)),
        pl.BlockSpec((HD, 3 * HD), lambda i: (0, 0)),
        pl.BlockSpec((1, 3 * HD), lambda i: (0, 0)),
    ]
    out_shape = [jax.ShapeDtypeStruct((B, m, HD), jnp.float32)]
    out_specs = [pl.BlockSpec((B, tile_m, HD), lambda i: (0, i, 0))]
    if emit_transposed:
        out_shape.append(jax.ShapeDtypeStruct((m, N_ALL), jnp.bfloat16))
        out_specs.append(pl.BlockSpec((tile_m, N_ALL), lambda i: (i, 0)))

    def body(*refs):
        if emit_transposed:
            _stage_body(*refs, packed_h=packed_h)
        else:
            _stage_body(*refs, None, packed_h=packed_h)

    return pl.pallas_call(
        body,
        grid=grid,
        in_specs=in_specs,
        out_specs=out_specs,
        out_shape=out_shape,
        compiler_params=pltpu.CompilerParams(
            dimension_semantics=("parallel",),
            vmem_limit_bytes=64 * 1024 * 1024,
        ),
    )(adj, src, hprev, w1t, b1, w2t, b2, wiht, bih, whht, bhh)


def kernel(v_feats, c_feats, H, H_t, W1, b1, W2, b2,
           var_wih, var_whh, var_bih, var_bhh,
           chk_wih, chk_whh, chk_bih, chk_bhh):
    w1t = W1.T
    w2t = W2.T
    b1r = b1.reshape(1, HD)
    b2r = b2.reshape(1, HD)
    chk_wiht = chk_wih.T
    chk_whht = chk_whh.T
    var_wiht = var_wih.T
    var_whht = var_whh.T
    chk_bihr = chk_bih.reshape(1, 3 * HD)
    chk_bhhr = chk_bhh.reshape(1, 3 * HD)
    var_bihr = var_bih.reshape(1, 3 * HD)
    var_bhhr = var_bhh.reshape(1, 3 * HD)

    v_src = jnp.transpose(v_feats, (1, 0, 2)).reshape(-1, N_ALL).astype(jnp.bfloat16)
    c_new, c_newT = _stage(H, v_src, c_feats,
                           w1t, b1r, w2t, b2r,
                           chk_wiht, chk_bihr, chk_whht, chk_bhhr,
                           tile_m=512, emit_transposed=True, packed_h=False)
    (v_new,) = _stage(H_t, c_newT, v_feats,
                      w1t, b1r, w2t, b2r,
                      var_wiht, var_bihr, var_whht, var_bhhr,
                      tile_m=1024, emit_transposed=False, packed_h=False)
    return (v_new, c_new)
